# Initial kernel scaffold; baseline (speedup 1.0000x reference)
import jax
import jax.numpy as jnp
from jax.experimental import pallas as pl


def kernel(n, relative_attention_bias):
    raise NotImplementedError("write your pallas kernel here")


# speedup vs baseline: 41.2722x; 41.2722x over previous
import functools
import math

import jax
import jax.numpy as jnp
from jax import lax
from jax.experimental import pallas as pl
from jax.experimental.pallas import tpu as pltpu
from jax.experimental.pallas import tpu_sc as plsc

_N = 2048
_H = 16
_WLEN = 4096
_ROWS_PER_WORKER = _N * _H // 32
_GROUPS = _ROWS_PER_WORKER // 8
_DEPTH_G = 2


def _bucket_index(relative_position, num_buckets=32, max_distance=128):
    ret = 0
    nneg = -relative_position
    num_buckets //= 2
    ret += (nneg < 0).astype(jnp.int32) * num_buckets
    nn = jnp.abs(nneg)
    max_exact = num_buckets // 2
    is_small = nn < max_exact
    val_if_large = max_exact + (
        jnp.log(nn.astype(jnp.float32) / max_exact)
        / math.log(max_distance / max_exact)
        * (num_buckets - max_exact)
    ).astype(jnp.int32)
    val_if_large = jnp.minimum(val_if_large, jnp.full_like(val_if_large, num_buckets - 1))
    ret += jnp.where(is_small, nn, val_if_large)
    return ret


@functools.partial(
    pl.kernel,
    mesh=plsc.VectorSubcoreMesh(core_axis_name="c", subcore_axis_name="s"),
    out_type=jax.ShapeDtypeStruct((_H * _N * _N,), jnp.float32),
    compiler_params=pltpu.CompilerParams(needs_layout_passes=False),
    scratch_types=[
        pltpu.VMEM((32 * _H,), jnp.float32),
        pltpu.VMEM((8 * _WLEN,), jnp.int32),
        pltpu.VMEM((_WLEN,), jnp.float32),
        pltpu.VMEM((_WLEN,), jnp.float32),
        pltpu.VMEM((_WLEN,), jnp.float32),
        pltpu.VMEM((_WLEN,), jnp.float32),
        pltpu.VMEM((_WLEN,), jnp.float32),
        pltpu.VMEM((_WLEN,), jnp.float32),
        pltpu.VMEM((_WLEN,), jnp.float32),
        pltpu.VMEM((_WLEN,), jnp.float32),
        pltpu.SemaphoreType.DMA,
    ],
)
def _rpb_sc_kernel(table_hbm, bidx_hbm, out_hbm, table_v, bidx_v,
                   r0, r1, r2, r3, r4, r5, r6, r7, sem):
    rev = [r0, r1, r2, r3, r4, r5, r6, r7]
    c = lax.axis_index("c")
    s = lax.axis_index("s")
    h = s
    i_base = c * _ROWS_PER_WORKER

    pltpu.sync_copy(table_hbm, table_v)
    pltpu.sync_copy(bidx_hbm, bidx_v)

    def gather_body(k, carry):
        base = k * 16
        for s8 in range(8):
            idx16 = bidx_v[pl.ds(s8 * _WLEN + base, 16)]
            vals = plsc.load_gather(table_v, [idx16 + h])
            rev[s8][pl.ds(base, 16)] = vals
        return carry

    lax.fori_loop(0, _WLEN // 16, gather_body, 0)

    def fire_group(u):
        for r in range(8):
            i = i_base + u * 8 + r
            s8 = 7 - r
            q = pl.multiple_of((_N - 1) - i - s8, 8)
            off = pl.multiple_of((h * _N + i) * _N, 8)
            pltpu.async_copy(
                rev[s8].at[pl.ds(q, _N)], out_hbm.at[pl.ds(off, _N)], sem
            )

    def drain_one():
        pltpu.make_async_copy(
            out_hbm.at[pl.ds(0, _N)], r0.at[pl.ds(0, _N)], sem
        ).wait()

    for u in range(_DEPTH_G):
        fire_group(u)

    def dma_body(u, carry):
        for _ in range(8):
            drain_one()
        fire_group(u + _DEPTH_G)
        return carry

    lax.fori_loop(0, _GROUPS - _DEPTH_G, dma_body, 0)

    for _ in range(8 * _DEPTH_G):
        drain_one()


def kernel(n, relative_attention_bias):
    table = relative_attention_bias.astype(jnp.float32)
    n_zero = (jnp.asarray(n) * 0).astype(jnp.int32)
    x = jnp.arange(_WLEN + 7, dtype=jnp.int32) + n_zero
    rel = jnp.maximum((_N - 1) - x, jnp.int32(-(_N - 1)))
    widx = _bucket_index(rel) * _H
    bidx8 = jnp.concatenate(
        [lax.dynamic_slice(widx, (s,), (_WLEN,)) for s in range(8)])
    out = _rpb_sc_kernel(table.reshape(-1), bidx8.astype(jnp.int32))
    return out.reshape(_H, _N, _N)


# speedup vs baseline: 42.3350x; 1.0258x over previous
import functools
import math

import jax
import jax.numpy as jnp
from jax import lax
from jax.experimental import pallas as pl
from jax.experimental.pallas import tpu as pltpu
from jax.experimental.pallas import tpu_sc as plsc

_N = 2048
_H = 16
_WLEN = 4096
_ROWS_PER_WORKER = _N * _H // 32
_GROUPS = _ROWS_PER_WORKER // 8
_DEPTH_G = 4


def _bucket_index(relative_position, num_buckets=32, max_distance=128):
    ret = 0
    nneg = -relative_position
    num_buckets //= 2
    ret += (nneg < 0).astype(jnp.int32) * num_buckets
    nn = jnp.abs(nneg)
    max_exact = num_buckets // 2
    is_small = nn < max_exact
    val_if_large = max_exact + (
        jnp.log(nn.astype(jnp.float32) / max_exact)
        / math.log(max_distance / max_exact)
        * (num_buckets - max_exact)
    ).astype(jnp.int32)
    val_if_large = jnp.minimum(val_if_large, jnp.full_like(val_if_large, num_buckets - 1))
    ret += jnp.where(is_small, nn, val_if_large)
    return ret


@functools.partial(
    pl.kernel,
    mesh=plsc.VectorSubcoreMesh(core_axis_name="c", subcore_axis_name="s"),
    out_type=jax.ShapeDtypeStruct((_H, _N, _N), jnp.float32),
    compiler_params=pltpu.CompilerParams(
        needs_layout_passes=False, use_tc_tiling_on_sc=False),
    scratch_types=[
        pltpu.VMEM((32 * _H,), jnp.float32),
        pltpu.VMEM((8 * _WLEN,), jnp.int32),
        pltpu.VMEM((8, _WLEN), jnp.float32),
        pltpu.SemaphoreType.DMA,
    ],
)
def _rpb_sc_kernel(table_hbm, bidx_hbm, out_hbm, table_v, bidx_v, rev2, sem):
    c = lax.axis_index("c")
    s = lax.axis_index("s")
    h = s
    i_base = c * _ROWS_PER_WORKER

    pltpu.sync_copy(table_hbm, table_v)
    pltpu.sync_copy(bidx_hbm, bidx_v)

    def gather_body(k, carry):
        base = k * 16
        for r in range(8):
            idx16 = bidx_v[pl.ds(r * _WLEN + base, 16)]
            vals = plsc.load_gather(table_v, [idx16 + h])
            rev2[r, pl.ds(base, 16)] = vals
        return carry

    lax.fori_loop(0, _WLEN // 16, gather_body, 0)

    def fire_group(u):
        q = pl.multiple_of(2040 - i_base - u * 8, 8)
        i0 = pl.multiple_of(i_base + u * 8, 8)
        pltpu.async_copy(
            rev2.at[:, pl.ds(q, _N)], out_hbm.at[h, pl.ds(i0, 8), :], sem
        )

    def drain_one():
        pltpu.make_async_copy(
            out_hbm.at[0, pl.ds(0, 8), :], rev2.at[:, pl.ds(0, _N)], sem
        ).wait()

    for u in range(_DEPTH_G):
        fire_group(u)

    def dma_body(u, carry):
        drain_one()
        fire_group(u + _DEPTH_G)
        return carry

    lax.fori_loop(0, _GROUPS - _DEPTH_G, dma_body, 0)

    for _ in range(_DEPTH_G):
        drain_one()


def kernel(n, relative_attention_bias):
    table = relative_attention_bias.astype(jnp.float32)
    n_zero = (jnp.asarray(n) * 0).astype(jnp.int32)
    x = jnp.arange(_WLEN + 7, dtype=jnp.int32) + n_zero
    rel = jnp.maximum((_N - 1) - x, jnp.int32(-(_N - 1)))
    widx = _bucket_index(rel) * _H
    bidx8 = jnp.concatenate(
        [lax.dynamic_slice(widx, (7 - r,), (_WLEN,)) for r in range(8)])
    return _rpb_sc_kernel(table.reshape(-1), bidx8.astype(jnp.int32))


# speedup vs baseline: 94.7175x; 2.2373x over previous
import functools
import math

import jax
import jax.numpy as jnp
from jax import lax
from jax.experimental import pallas as pl
from jax.experimental.pallas import tpu as pltpu
from jax.experimental.pallas import tpu_sc as plsc

_N = 2048
_H = 16
_BW = 4096
_RW = 4224
_XW = _RW + 7


def _bucket_index(relative_position, num_buckets=32, max_distance=128):
    ret = 0
    nneg = -relative_position
    num_buckets //= 2
    ret += (nneg < 0).astype(jnp.int32) * num_buckets
    nn = jnp.abs(nneg)
    max_exact = num_buckets // 2
    is_small = nn < max_exact
    val_if_large = max_exact + (
        jnp.log(nn.astype(jnp.float32) / max_exact)
        / math.log(max_distance / max_exact)
        * (num_buckets - max_exact)
    ).astype(jnp.int32)
    val_if_large = jnp.minimum(val_if_large, jnp.full_like(val_if_large, num_buckets - 1))
    ret += jnp.where(is_small, nn, val_if_large)
    return ret


@functools.partial(
    pl.kernel,
    mesh=plsc.VectorSubcoreMesh(core_axis_name="c", subcore_axis_name="s"),
    out_type=jax.ShapeDtypeStruct((_H, 16, 8, _BW), jnp.float32),
    compiler_params=pltpu.CompilerParams(
        needs_layout_passes=False, use_tc_tiling_on_sc=False),
    scratch_types=[
        pltpu.VMEM((32 * _H,), jnp.float32),
        pltpu.VMEM((8 * _RW,), jnp.int32),
        pltpu.VMEM((8, _RW), jnp.float32),
        pltpu.SemaphoreType.DMA,
    ],
)
def _gather_sc_kernel(table_hbm, bidx_hbm, b_hbm, table_v, bidx_v, revr_v, sem):
    c = lax.axis_index("c")
    s = lax.axis_index("s")
    h = s

    pltpu.sync_copy(table_hbm, table_v)
    pltpu.sync_copy(bidx_hbm, bidx_v)

    def gather_body(k, carry):
        base = k * 16
        for r in range(8):
            idx16 = bidx_v[pl.ds(r * _RW + base, 16)]
            vals = plsc.load_gather(table_v, [idx16 + h])
            revr_v[r, pl.ds(base, 16)] = vals
        return carry

    lax.fori_loop(0, _RW // 16, gather_body, 0)

    for si in range(8):
        sh = c * 8 + si
        q = pl.multiple_of(120 - 8 * sh, 8)
        pltpu.async_copy(revr_v.at[:, pl.ds(q, _BW)], b_hbm.at[h, sh], sem)
    for _ in range(8):
        pltpu.make_async_copy(
            b_hbm.at[0, 0], revr_v.at[:, pl.ds(0, _BW)], sem
        ).wait()


def _materialize_tc_body(b_ref, out_ref):
    for k in range(_N // 8):
        s_ = k % 16
        m = k // 16
        off = 128 * (15 - m)
        out_ref[0, pl.ds(8 * k, 8), :] = b_ref[0, s_, :, pl.ds(off, _N)]


def kernel(n, relative_attention_bias):
    table = relative_attention_bias.astype(jnp.float32)
    n_zero = (jnp.asarray(n) * 0).astype(jnp.int32)
    x = jnp.arange(_XW, dtype=jnp.int32) + n_zero
    rel = jnp.maximum((_N - 1) - x, jnp.int32(-(_N - 1)))
    widx = _bucket_index(rel) * _H
    bidx = jnp.concatenate(
        [lax.dynamic_slice(widx, (7 - r,), (_RW,)) for r in range(8)])

    b = _gather_sc_kernel(table.reshape(-1), bidx.astype(jnp.int32))

    return pl.pallas_call(
        _materialize_tc_body,
        out_shape=jax.ShapeDtypeStruct((_H, _N, _N), jnp.float32),
        grid=(_H,),
        in_specs=[pl.BlockSpec((1, 16, 8, _BW), lambda hh: (hh, 0, 0, 0))],
        out_specs=pl.BlockSpec((1, _N, _N), lambda hh: (hh, 0, 0)),
        compiler_params=pltpu.CompilerParams(
            dimension_semantics=("arbitrary",)),
    )(b)


# speedup vs baseline: 98.1875x; 1.0366x over previous
import functools
import math

import jax
import jax.numpy as jnp
from jax import lax
from jax.experimental import pallas as pl
from jax.experimental.pallas import tpu as pltpu
from jax.experimental.pallas import tpu_sc as plsc

_N = 2048
_H = 16
_BW = 4096
_RW = 4224
_XW = _RW + 7
_DEPTH = 16


def _bucket_index(relative_position, num_buckets=32, max_distance=128):
    ret = 0
    nneg = -relative_position
    num_buckets //= 2
    ret += (nneg < 0).astype(jnp.int32) * num_buckets
    nn = jnp.abs(nneg)
    max_exact = num_buckets // 2
    is_small = nn < max_exact
    val_if_large = max_exact + (
        jnp.log(nn.astype(jnp.float32) / max_exact)
        / math.log(max_distance / max_exact)
        * (num_buckets - max_exact)
    ).astype(jnp.int32)
    val_if_large = jnp.minimum(val_if_large, jnp.full_like(val_if_large, num_buckets - 1))
    ret += jnp.where(is_small, nn, val_if_large)
    return ret


@functools.partial(
    pl.kernel,
    mesh=plsc.VectorSubcoreMesh(core_axis_name="c", subcore_axis_name="s"),
    out_type=jax.ShapeDtypeStruct((_H, 16, _BW // 16, 128), jnp.float32),
    compiler_params=pltpu.CompilerParams(
        needs_layout_passes=False, use_tc_tiling_on_sc=False),
    scratch_types=[
        pltpu.VMEM((32 * _H,), jnp.float32),
        pltpu.VMEM((8 * _RW,), jnp.int32),
        pltpu.VMEM((8, _RW), jnp.float32),
        pltpu.SemaphoreType.DMA,
    ],
)
def _gather_sc_kernel(table_hbm, bidx_hbm, b_hbm, table_v, bidx_v, revr_v, sem):
    c = lax.axis_index("c")
    s = lax.axis_index("s")
    h = s

    pltpu.sync_copy(table_hbm, table_v)
    pltpu.sync_copy(bidx_hbm, bidx_v)

    def gather_body(k, carry):
        base = k * 16
        for r in range(8):
            idx16 = bidx_v[pl.ds(r * _RW + base, 16)]
            vals = plsc.load_gather(table_v, [idx16 + h])
            revr_v[r, pl.ds(base, 16)] = vals
        return carry

    lax.fori_loop(0, _RW // 16, gather_body, 0)

    def fire(si, tj):
        sh = c * 8 + si
        q = pl.multiple_of(120 - 8 * sh + 128 * tj, 8)
        pltpu.async_copy(
            revr_v.at[:, pl.ds(q, 128)],
            b_hbm.at[h, sh, pl.ds(8 * tj, 8), :],
            sem,
        )

    def drain_one():
        pltpu.make_async_copy(
            b_hbm.at[0, 0, pl.ds(0, 8), :], revr_v.at[:, pl.ds(0, 128)], sem
        ).wait()

    n_tiles = _BW // 128
    for tj in range(2):
        for si in range(8):
            fire(si, tj)

    def dma_body(tj, carry):
        for _ in range(8):
            drain_one()
        for si in range(8):
            fire(si, tj + 2)
        return carry

    lax.fori_loop(0, n_tiles - 2, dma_body, 0)
    for _ in range(16):
        drain_one()


def _materialize_tc_body(b_ref, out_ref):
    def body(k, carry):
        s_ = k % 16
        m = k // 16
        tj0 = 15 - m
        for tj in range(16):
            out_ref[0, pl.ds(8 * k, 8), 128 * tj:128 * (tj + 1)] = (
                b_ref[0, s_, pl.ds(8 * (tj0 + tj), 8), :]
            )
        return carry

    lax.fori_loop(0, _N // 8, body, 0)


def kernel(n, relative_attention_bias):
    table = relative_attention_bias.astype(jnp.float32)
    n_zero = (jnp.asarray(n) * 0).astype(jnp.int32)
    x = jnp.arange(_XW, dtype=jnp.int32) + n_zero
    rel = jnp.maximum((_N - 1) - x, jnp.int32(-(_N - 1)))
    widx = _bucket_index(rel) * _H
    bidx = jnp.concatenate(
        [lax.dynamic_slice(widx, (7 - r,), (_RW,)) for r in range(8)])

    b = _gather_sc_kernel(table.reshape(-1), bidx.astype(jnp.int32))

    return pl.pallas_call(
        _materialize_tc_body,
        out_shape=jax.ShapeDtypeStruct((_H, _N, _N), jnp.float32),
        grid=(_H,),
        in_specs=[pl.BlockSpec(
            (1, 16, _BW // 16, 128), lambda hh: (hh, 0, 0, 0))],
        out_specs=pl.BlockSpec((1, _N, _N), lambda hh: (hh, 0, 0)),
        compiler_params=pltpu.CompilerParams(
            dimension_semantics=("arbitrary",)),
    )(b)


# speedup vs baseline: 98.2003x; 1.0001x over previous
import functools
import math

import jax
import jax.numpy as jnp
from jax import lax
from jax.experimental import pallas as pl
from jax.experimental.pallas import tpu as pltpu
from jax.experimental.pallas import tpu_sc as plsc

_N = 2048
_H = 16
_BW = 4096
_RW = 4224
_XW = _RW + 7
_DEPTH = 16


def _bucket_index(relative_position, num_buckets=32, max_distance=128):
    ret = 0
    nneg = -relative_position
    num_buckets //= 2
    ret += (nneg < 0).astype(jnp.int32) * num_buckets
    nn = jnp.abs(nneg)
    max_exact = num_buckets // 2
    is_small = nn < max_exact
    val_if_large = max_exact + (
        jnp.log(nn.astype(jnp.float32) / max_exact)
        / math.log(max_distance / max_exact)
        * (num_buckets - max_exact)
    ).astype(jnp.int32)
    val_if_large = jnp.minimum(val_if_large, jnp.full_like(val_if_large, num_buckets - 1))
    ret += jnp.where(is_small, nn, val_if_large)
    return ret


@functools.partial(
    pl.kernel,
    mesh=plsc.VectorSubcoreMesh(core_axis_name="c", subcore_axis_name="s"),
    out_type=jax.ShapeDtypeStruct((_H, 16, _BW // 16, 128), jnp.float32),
    compiler_params=pltpu.CompilerParams(
        needs_layout_passes=False, use_tc_tiling_on_sc=False),
    scratch_types=[
        pltpu.VMEM((32 * _H,), jnp.float32),
        pltpu.VMEM((8 * _RW,), jnp.int32),
        pltpu.VMEM((8, _RW), jnp.float32),
        pltpu.SemaphoreType.DMA,
    ],
)
def _gather_sc_kernel(table_hbm, bidx_hbm, b_hbm, table_v, bidx_v, revr_v, sem):
    c = lax.axis_index("c")
    s = lax.axis_index("s")
    h = s

    pltpu.sync_copy(table_hbm, table_v)
    pltpu.sync_copy(bidx_hbm, bidx_v)

    h32 = h * 32
    def gather_body(k, carry):
        base = k * 16
        for r in range(8):
            idx16 = bidx_v[pl.ds(r * _RW + base, 16)]
            vals = plsc.load_gather(table_v, [idx16 + h32])
            revr_v[r, pl.ds(base, 16)] = vals
        return carry

    lax.fori_loop(0, _RW // 16, gather_body, 0)

    def fire(si, tj):
        sh = c * 8 + si
        q = pl.multiple_of(120 - 8 * sh + 128 * tj, 8)
        pltpu.async_copy(
            revr_v.at[:, pl.ds(q, 128)],
            b_hbm.at[h, sh, pl.ds(8 * tj, 8), :],
            sem,
        )

    def drain_one():
        pltpu.make_async_copy(
            b_hbm.at[0, 0, pl.ds(0, 8), :], revr_v.at[:, pl.ds(0, 128)], sem
        ).wait()

    n_tiles = _BW // 128
    for tj in range(2):
        for si in range(8):
            fire(si, tj)

    def dma_body(tj, carry):
        for _ in range(8):
            drain_one()
        for si in range(8):
            fire(si, tj + 2)
        return carry

    lax.fori_loop(0, n_tiles - 2, dma_body, 0)
    for _ in range(16):
        drain_one()


def _materialize_tc_body(b_ref, out_ref):
    def body(k, carry):
        s_ = k % 16
        m = k // 16
        tj0 = 15 - m
        for tj in range(16):
            out_ref[0, pl.ds(8 * k, 8), 128 * tj:128 * (tj + 1)] = (
                b_ref[0, s_, pl.ds(8 * (tj0 + tj), 8), :]
            )
        return carry

    lax.fori_loop(0, _N // 8, body, 0)


def kernel(n, relative_attention_bias):
    table = relative_attention_bias.astype(jnp.float32)
    n_zero = (jnp.asarray(n) * 0).astype(jnp.int32)
    x = jnp.arange(_XW, dtype=jnp.int32) + n_zero
    rel = jnp.maximum((_N - 1) - x, jnp.int32(-(_N - 1)))
    widx = _bucket_index(rel)
    bidx = jnp.concatenate(
        [lax.dynamic_slice(widx, (7 - r,), (_RW,)) for r in range(8)])

    b = _gather_sc_kernel(table.T.reshape(-1), bidx.astype(jnp.int32))

    return pl.pallas_call(
        _materialize_tc_body,
        out_shape=jax.ShapeDtypeStruct((_H, _N, _N), jnp.float32),
        grid=(_H,),
        in_specs=[pl.BlockSpec(
            (1, 16, _BW // 16, 128), lambda hh: (hh, 0, 0, 0))],
        out_specs=pl.BlockSpec((1, _N, _N), lambda hh: (hh, 0, 0)),
        compiler_params=pltpu.CompilerParams(
            dimension_semantics=("arbitrary",)),
    )(b)
